# baseline (speedup 1.0000x reference)
import functools
import math

import jax
import jax.numpy as jnp
from jax import lax
from jax.experimental import pallas as pl
from jax.experimental.pallas import tpu as pltpu
from jax.experimental.pallas import tpu_sc as plsc

D_MODEL = 64
PAIR_W = 2 * D_MODEL
SCALE = math.sqrt(D_MODEL)
LANES = 16
NBUF = 2
CHUNK = 160


@functools.lru_cache(maxsize=None)
def _make_emb(B, V):
    info = plsc.get_sparse_core_info()
    nw = info.num_cores * info.num_subcores
    b_per_w = B // nw
    n_chunks = b_per_w // CHUNK
    assert b_per_w % CHUNK == 0 and n_chunks % NBUF == 0
    mesh = plsc.VectorSubcoreMesh(core_axis_name="c", subcore_axis_name="s")

    @functools.partial(
        pl.kernel,
        mesh=mesh,
        out_type=jax.ShapeDtypeStruct((B // 2, PAIR_W), jnp.float32),
        scratch_types=[
            pltpu.VMEM((b_per_w,), jnp.int32),
            pltpu.VMEM((b_per_w,), jnp.int32),
            pltpu.VMEM((CHUNK // 2, PAIR_W), jnp.float32),
            *[pltpu.VMEM((CHUNK, PAIR_W), jnp.float32) for _ in range(NBUF)],
            *[pltpu.SemaphoreType.DMA for _ in range(NBUF)],
        ],
        compiler_params=pltpu.CompilerParams(needs_layout_passes=False),
    )
    def emb(x_hbm, table_hbm, out_hbm, idx_v, half_v, obuf, *bufs_sems):
        bufs = bufs_sems[:NBUF]
        sems = bufs_sems[NBUF:]
        wid = lax.axis_index("s") * info.num_cores + lax.axis_index("c")
        base = wid * b_per_w

        pltpu.sync_copy(x_hbm.at[pl.ds(pl.multiple_of(base, 128), b_per_w)], idx_v)

        def halve(i, c2):
            sl = pl.ds(i * LANES, LANES)
            half_v[sl] = lax.shift_right_logical(idx_v[sl], 1)
            return c2

        lax.fori_loop(0, b_per_w // LANES, halve, 0)

        def idx_list(j):
            return half_v.at[pl.ds(j * CHUNK, CHUNK)]

        for b in range(NBUF):
            pltpu.async_copy(table_hbm.at[idx_list(b)], bufs[b], sems[b])

        lane = lax.iota(jnp.int32, 16)

        def group_body(g, carry):
            for b in range(NBUF):
                j = g * NBUF + b
                buf = bufs[b]
                pltpu.make_async_copy(
                    table_hbm.at[idx_list(j)], buf, sems[b]
                ).wait()

                def fix_pair(u, c2):
                    for h in range(2):
                        t = 2 * u + h
                        t16 = jnp.full((16,), t, jnp.int32)
                        raw = plsc.load_gather(
                            idx_v, [jnp.full((16,), j * CHUNK + t, jnp.int32)]
                        )
                        col = (raw & 1) * D_MODEL + lane
                        for c in range(D_MODEL // LANES):
                            v = plsc.load_gather(buf, [t16, col + (c * LANES)])
                            obuf[u, pl.ds(h * D_MODEL + c * LANES, LANES)] = (
                                v * SCALE
                            )
                    return c2

                lax.fori_loop(0, CHUNK // 2, fix_pair, 0)

                pltpu.sync_copy(
                    obuf,
                    out_hbm.at[
                        pl.ds(
                            pl.multiple_of((base + j * CHUNK) // 2, 8),
                            CHUNK // 2,
                        )
                    ],
                )

                @pl.when(j + NBUF < n_chunks)
                def _():
                    pltpu.async_copy(
                        table_hbm.at[idx_list(j + NBUF)], buf, sems[b]
                    )

            return carry

        lax.fori_loop(0, n_chunks // NBUF, group_body, 0)

    return emb


def kernel(x, table):
    B = x.shape[0] * x.shape[1]
    x1 = x.reshape(B)
    t2 = table.reshape(table.shape[0] // 2, PAIR_W)
    o = _make_emb(B, table.shape[0])(x1, t2)
    return o.reshape(x.shape + (D_MODEL,))
